# baseline (speedup 1.0000x reference)
import functools
import jax
import jax.numpy as jnp
import numpy as np
from jax import lax
from jax.experimental import pallas as pl
from jax.experimental.pallas import tpu as pltpu
from jax.experimental.pallas import tpu_sc as plsc

GRID = (64, 64, 64)
F = 256
H = 8
D = 32
S27 = 27
NFF = 1024
_EPS = 1e-5
_INVSQRT_D = 1.0 / np.sqrt(D).astype(np.float32)


def _ln(x, g, b):
    mu = jnp.mean(x, axis=-1, keepdims=True)
    xc = x - mu
    var = jnp.mean(xc * xc, axis=-1, keepdims=True)
    return xc * jax.lax.rsqrt(var + _EPS) * g + b


def _qkv_body(x_ref, wq_ref, wk_ref, bk_ref, wv_ref, bv_ref, g_ref, b_ref,
              q_ref, kn_ref, vn_ref):
    x = x_ref[...]
    dn = (((1,), (1,)), ((), ()))
    q_ref[...] = jax.lax.dot_general(x, wq_ref[...], dn,
                                     preferred_element_type=jnp.float32, precision=jax.lax.Precision.HIGHEST)
    k = jax.lax.dot_general(x, wk_ref[...], dn,
                            preferred_element_type=jnp.float32, precision=jax.lax.Precision.HIGHEST) + bk_ref[...]
    v = jax.lax.dot_general(x, wv_ref[...], dn,
                            preferred_element_type=jnp.float32, precision=jax.lax.Precision.HIGHEST) + bv_ref[...]
    g = g_ref[...]
    b = b_ref[...]
    kn_ref[...] = _ln(k, g, b)
    vn_ref[...] = _ln(v, g, b)


def _attn_mlp_body(q_ref, kh_ref, vh_ref, vm_ref, ic_ref, x_ref,
                   g_ref, b_ref, w1_ref, b1_ref, w2_ref, b2_ref, y_ref,
                   *, blk):
    B = blk
    q = q_ref[...]
    kh = kh_ref[...]
    vh = vh_ref[...]
    vm = vm_ref[...]

    fi = jax.lax.broadcasted_iota(jnp.int32, (F, H), 0)
    hi = jax.lax.broadcasted_iota(jnp.int32, (F, H), 1)
    seg = (fi // D == hi).astype(jnp.float32)

    qb = jnp.reshape(q[:, None, :] * jnp.ones((1, S27, 1), jnp.float32),
                     (B * S27, F))
    prod = qb * kh
    dots = jax.lax.dot_general(prod, seg, (((1,), (0,)), ((), ())),
                               preferred_element_type=jnp.float32, precision=jax.lax.Precision.HIGHEST)
    dots = dots * _INVSQRT_D
    dots = jnp.where(vm > 0.5, dots, -1e30)
    d3 = jnp.reshape(dots, (B, S27, H))
    m = jnp.max(d3, axis=1, keepdims=True)
    e = jnp.exp(d3 - m)
    z = jnp.sum(e, axis=1, keepdims=True)
    a3 = e / z

    a2 = jnp.reshape(a3, (B * S27, H))
    aw = jax.lax.dot_general(a2, seg, (((1,), (1,)), ((), ())),
                             preferred_element_type=jnp.float32, precision=jax.lax.Precision.HIGHEST)
    out = jnp.sum(jnp.reshape(aw * vh, (B, S27, F)), axis=1)

    ic = ic_ref[...][:, 0:1]
    x = x_ref[...] + out * ic

    h = _ln(x, g_ref[...], b_ref[...])
    h = jax.lax.dot_general(h, w1_ref[...], (((1,), (1,)), ((), ())),
                            preferred_element_type=jnp.float32, precision=jax.lax.Precision.HIGHEST) + b1_ref[...]
    h = 0.5 * h * (1.0 + jax.lax.erf(h * np.float32(1.0 / np.sqrt(2.0))))
    h = jax.lax.dot_general(h, w2_ref[...], (((1,), (1,)), ((), ())),
                            preferred_element_type=jnp.float32, precision=jax.lax.Precision.HIGHEST) + b2_ref[...]
    y_ref[...] = x + h


def _full(shape):
    return pl.BlockSpec(shape, lambda i: (0,) * len(shape))


def _sc_gather2(kn, vn, idxf):
    NT = kn.shape[0]
    M = idxf.shape[0]
    NW = 32
    per_w = M // NW
    C = 96
    nchunk = per_w // C
    mesh = plsc.VectorSubcoreMesh(core_axis_name="c", subcore_axis_name="s")

    @functools.partial(
        pl.kernel, mesh=mesh,
        out_type=[jax.ShapeDtypeStruct((M, F), jnp.float32)] * 2,
        scratch_types=[
            pltpu.VMEM((per_w,), jnp.int32),
            pltpu.VMEM((2, C, F), jnp.float32),
            pltpu.VMEM((2, C, F), jnp.float32),
            pltpu.SemaphoreType.DMA,
            pltpu.SemaphoreType.DMA,
            pltpu.SemaphoreType.DMA,
            pltpu.SemaphoreType.DMA,
        ],
    )
    def body(kn_hbm, vn_hbm, idx_hbm, kh_hbm, vh_hbm,
             idx_v, kbuf, vbuf, ks0, ks1, vs0, vs1):
        wid = lax.axis_index("s") * 2 + lax.axis_index("c")
        base = wid * per_w
        pltpu.sync_copy(idx_hbm.at[pl.ds(base, per_w)], idx_v)
        ksems = (ks0, ks1)
        vsems = (vs0, vs1)

        def start(c, b):
            isl = idx_v.at[pl.ds(c * C, C)]
            pltpu.make_async_copy(kn_hbm.at[isl], kbuf.at[b], ksems[b]).start()
            pltpu.make_async_copy(vn_hbm.at[isl], vbuf.at[b], vsems[b]).start()

        def drain(c, b):
            isl = idx_v.at[pl.ds(c * C, C)]
            pltpu.make_async_copy(kn_hbm.at[isl], kbuf.at[b], ksems[b]).wait()
            pltpu.make_async_copy(vn_hbm.at[isl], vbuf.at[b], vsems[b]).wait()
            pltpu.sync_copy(kbuf.at[b], kh_hbm.at[pl.ds(base + c * C, C)])
            pltpu.sync_copy(vbuf.at[b], vh_hbm.at[pl.ds(base + c * C, C)])

        start(0, 0)
        start(1, 1)

        @pl.loop(0, nchunk, step=2)
        def _(g):
            for b in range(2):
                c = g + b
                drain(c, b)

                @pl.when(c + 2 < nchunk)
                def _():
                    start(c + 2, b)

    return body(kn, vn, idxf)


def kernel(coords, feats, Wq, Wk, bk, Wv, bv, n1g, n1b, n2g, n2b, W1, b1, W2, b2):
    n = feats.shape[0]
    B = 128
    N = ((n + 511) // 512) * 512

    ids = jnp.arange(1, n + 1, dtype=jnp.int32)
    dense = jnp.zeros(GRID, jnp.int32).at[
        coords[:, 0], coords[:, 1], coords[:, 2]].max(ids)
    padded = jnp.pad(dense, 1)
    offs = jnp.arange(27)
    di, dj, dk = offs // 9, (offs // 3) % 3, offs % 3
    hit = padded[coords[:, 0:1] + di[None],
                 coords[:, 1:2] + dj[None],
                 coords[:, 2:3] + dk[None]] - 1

    hh = jnp.arange(H)
    Mk = jnp.zeros((H, H, D, D), Wk.dtype).at[hh, hh].set(Wk)
    Wkbd = Mk.transpose(0, 2, 1, 3).reshape(F, F)
    Mv = jnp.zeros((H, H, D, D), Wv.dtype).at[hh, hh].set(Wv)
    Wvbd = Mv.transpose(0, 2, 1, 3).reshape(F, F)
    bkf = bk.reshape(1, F)
    bvf = bv.reshape(1, F)

    xp = jnp.zeros((N, F), jnp.float32).at[:n].set(feats)

    Bq = 512
    q, kn, vn = pl.pallas_call(
        _qkv_body,
        grid=(N // Bq,),
        in_specs=[
            pl.BlockSpec((Bq, F), lambda i: (i, 0)),
            _full((F, F)), _full((F, F)), _full((1, F)),
            _full((F, F)), _full((1, F)),
            _full((1, F)), _full((1, F)),
        ],
        out_specs=[pl.BlockSpec((Bq, F), lambda i: (i, 0))] * 3,
        out_shape=[jax.ShapeDtypeStruct((N, F), jnp.float32)] * 3,
    )(xp, Wq, Wkbd, bkf, Wvbd, bvf, n1g.reshape(1, F), n1b.reshape(1, F))

    hitp = jnp.full((N, S27), -1, jnp.int32).at[:n].set(hit)
    idxf = jnp.maximum(hitp, 0).reshape(-1)
    kh2, vh2 = _sc_gather2(kn, vn, idxf)
    vm = jnp.broadcast_to(
        (hitp >= 0).astype(jnp.float32).reshape(N * S27, 1), (N * S27, 8))
    scp = hitp[:, 13]
    qsc = q[jnp.maximum(scp, 0)]
    ic8 = jnp.broadcast_to(
        (scp == jnp.arange(N)).astype(jnp.float32)[:, None], (N, 8))

    y = pl.pallas_call(
        functools.partial(_attn_mlp_body, blk=B),
        grid=(N // B,),
        in_specs=[
            pl.BlockSpec((B, F), lambda i: (i, 0)),
            pl.BlockSpec((B * S27, F), lambda i: (i, 0)),
            pl.BlockSpec((B * S27, F), lambda i: (i, 0)),
            pl.BlockSpec((B * S27, 8), lambda i: (i, 0)),
            pl.BlockSpec((B, 8), lambda i: (i, 0)),
            pl.BlockSpec((B, F), lambda i: (i, 0)),
            _full((1, F)), _full((1, F)),
            _full((NFF, F)), _full((1, NFF)),
            _full((F, NFF)), _full((1, F)),
        ],
        out_specs=pl.BlockSpec((B, F), lambda i: (i, 0)),
        out_shape=jax.ShapeDtypeStruct((N, F), jnp.float32),
    )(qsc, kh2, vh2, vm, ic8, xp,
      n2g.reshape(1, F), n2b.reshape(1, F),
      W1, b1.reshape(1, NFF), W2, b2.reshape(1, F))

    return y[:n]


# speedup vs baseline: 1.0007x; 1.0007x over previous
import functools
import jax
import jax.numpy as jnp
import numpy as np
from jax import lax
from jax.experimental import pallas as pl
from jax.experimental.pallas import tpu as pltpu
from jax.experimental.pallas import tpu_sc as plsc

GRID = (64, 64, 64)
F = 256
H = 8
D = 32
S27 = 27
NFF = 1024
_EPS = 1e-5
_INVSQRT_D = 1.0 / np.sqrt(D).astype(np.float32)


def _ln(x, g, b):
    mu = jnp.mean(x, axis=-1, keepdims=True)
    xc = x - mu
    var = jnp.mean(xc * xc, axis=-1, keepdims=True)
    return xc * jax.lax.rsqrt(var + _EPS) * g + b


def _qkv_body(x_ref, wq_ref, wk_ref, bk_ref, wv_ref, bv_ref, g_ref, b_ref,
              q_ref, kv_ref):
    x = x_ref[...]
    dn = (((1,), (1,)), ((), ()))
    q_ref[...] = jax.lax.dot_general(x, wq_ref[...], dn,
                                     preferred_element_type=jnp.float32, precision=jax.lax.Precision.HIGHEST)
    k = jax.lax.dot_general(x, wk_ref[...], dn,
                            preferred_element_type=jnp.float32, precision=jax.lax.Precision.HIGHEST) + bk_ref[...]
    v = jax.lax.dot_general(x, wv_ref[...], dn,
                            preferred_element_type=jnp.float32, precision=jax.lax.Precision.HIGHEST) + bv_ref[...]
    g = g_ref[...]
    b = b_ref[...]
    kv_ref[:, :F] = _ln(k, g, b)
    kv_ref[:, F:] = _ln(v, g, b)


def _attn_mlp_body(q_ref, kh_ref, vh_ref, vm_ref, ic_ref, x_ref,
                   g_ref, b_ref, w1_ref, b1_ref, w2_ref, b2_ref, y_ref,
                   *, blk):
    B = blk
    q = q_ref[...]
    kh = kh_ref[...]
    vh = vh_ref[...]
    vm = vm_ref[...]

    fi = jax.lax.broadcasted_iota(jnp.int32, (F, H), 0)
    hi = jax.lax.broadcasted_iota(jnp.int32, (F, H), 1)
    seg = (fi // D == hi).astype(jnp.float32)

    qb = jnp.reshape(q[:, None, :] * jnp.ones((1, S27, 1), jnp.float32),
                     (B * S27, F))
    prod = qb * kh
    dots = jax.lax.dot_general(prod, seg, (((1,), (0,)), ((), ())),
                               preferred_element_type=jnp.float32, precision=jax.lax.Precision.HIGHEST)
    dots = dots * _INVSQRT_D
    dots = jnp.where(vm > 0.5, dots, -1e30)
    d3 = jnp.reshape(dots, (B, S27, H))
    m = jnp.max(d3, axis=1, keepdims=True)
    e = jnp.exp(d3 - m)
    z = jnp.sum(e, axis=1, keepdims=True)
    a3 = e / z

    a2 = jnp.reshape(a3, (B * S27, H))
    aw = jax.lax.dot_general(a2, seg, (((1,), (1,)), ((), ())),
                             preferred_element_type=jnp.float32, precision=jax.lax.Precision.HIGHEST)
    out = jnp.sum(jnp.reshape(aw * vh, (B, S27, F)), axis=1)

    ic = ic_ref[...][:, 0:1]
    x = x_ref[...] + out * ic

    h = _ln(x, g_ref[...], b_ref[...])
    h = jax.lax.dot_general(h, w1_ref[...], (((1,), (1,)), ((), ())),
                            preferred_element_type=jnp.float32, precision=jax.lax.Precision.HIGHEST) + b1_ref[...]
    h = 0.5 * h * (1.0 + jax.lax.erf(h * np.float32(1.0 / np.sqrt(2.0))))
    h = jax.lax.dot_general(h, w2_ref[...], (((1,), (1,)), ((), ())),
                            preferred_element_type=jnp.float32, precision=jax.lax.Precision.HIGHEST) + b2_ref[...]
    y_ref[...] = x + h


def _full(shape):
    return pl.BlockSpec(shape, lambda i: (0,) * len(shape))


def _sc_gatherkv(kvn, idxf):
    M = idxf.shape[0]
    NW = 32
    per_w = M // NW
    C = 96
    nchunk = per_w // C
    F2 = 2 * F
    mesh = plsc.VectorSubcoreMesh(core_axis_name="c", subcore_axis_name="s")

    @functools.partial(
        pl.kernel, mesh=mesh,
        out_type=jax.ShapeDtypeStruct((M, F2), jnp.float32),
        scratch_types=[
            pltpu.VMEM((per_w,), jnp.int32),
            pltpu.VMEM((2, C, F2), jnp.float32),
            pltpu.SemaphoreType.DMA,
            pltpu.SemaphoreType.DMA,
            pltpu.SemaphoreType.DMA,
            pltpu.SemaphoreType.DMA,
        ],
    )
    def body(kv_hbm, idx_hbm, out_hbm, idx_v, buf, gs0, gs1, ss0, ss1):
        wid = lax.axis_index("s") * 2 + lax.axis_index("c")
        base = wid * per_w
        pltpu.sync_copy(idx_hbm.at[pl.ds(base, per_w)], idx_v)
        gsems = (gs0, gs1)
        ssems = (ss0, ss1)

        def gstart(c, b):
            isl = idx_v.at[pl.ds(c * C, C)]
            pltpu.make_async_copy(kv_hbm.at[isl], buf.at[b], gsems[b]).start()

        gstart(0, 0)
        gstart(1, 1)

        @pl.loop(0, nchunk, step=2)
        def _(g):
            for b in range(2):
                c = g + b
                isl = idx_v.at[pl.ds(c * C, C)]
                pltpu.make_async_copy(kv_hbm.at[isl], buf.at[b], gsems[b]).wait()
                st = pltpu.make_async_copy(
                    buf.at[b], out_hbm.at[pl.ds(base + c * C, C)], ssems[b])
                st.start()

                @pl.when(c + 2 < nchunk)
                def _():
                    pltpu.make_async_copy(
                        buf.at[b], out_hbm.at[pl.ds(base + c * C, C)],
                        ssems[b]).wait()
                    gstart(c + 2, b)

        pltpu.make_async_copy(
            buf.at[0], out_hbm.at[pl.ds(base, C)], ssems[0]).wait()
        pltpu.make_async_copy(
            buf.at[1], out_hbm.at[pl.ds(base, C)], ssems[1]).wait()

    return body(kvn, idxf)


def kernel(coords, feats, Wq, Wk, bk, Wv, bv, n1g, n1b, n2g, n2b, W1, b1, W2, b2):
    n = feats.shape[0]
    B = 128
    N = ((n + 511) // 512) * 512

    ids = jnp.arange(1, n + 1, dtype=jnp.int32)
    dense = jnp.zeros(GRID, jnp.int32).at[
        coords[:, 0], coords[:, 1], coords[:, 2]].max(ids)
    padded = jnp.pad(dense, 1)
    offs = jnp.arange(27)
    di, dj, dk = offs // 9, (offs // 3) % 3, offs % 3
    hit = padded[coords[:, 0:1] + di[None],
                 coords[:, 1:2] + dj[None],
                 coords[:, 2:3] + dk[None]] - 1

    hh = jnp.arange(H)
    Mk = jnp.zeros((H, H, D, D), Wk.dtype).at[hh, hh].set(Wk)
    Wkbd = Mk.transpose(0, 2, 1, 3).reshape(F, F)
    Mv = jnp.zeros((H, H, D, D), Wv.dtype).at[hh, hh].set(Wv)
    Wvbd = Mv.transpose(0, 2, 1, 3).reshape(F, F)
    bkf = bk.reshape(1, F)
    bvf = bv.reshape(1, F)

    xp = jnp.zeros((N, F), jnp.float32).at[:n].set(feats)

    Bq = 512
    q, kvn = pl.pallas_call(
        _qkv_body,
        grid=(N // Bq,),
        in_specs=[
            pl.BlockSpec((Bq, F), lambda i: (i, 0)),
            _full((F, F)), _full((F, F)), _full((1, F)),
            _full((F, F)), _full((1, F)),
            _full((1, F)), _full((1, F)),
        ],
        out_specs=[pl.BlockSpec((Bq, F), lambda i: (i, 0)),
                   pl.BlockSpec((Bq, 2 * F), lambda i: (i, 0))],
        out_shape=[jax.ShapeDtypeStruct((N, F), jnp.float32),
                   jax.ShapeDtypeStruct((N, 2 * F), jnp.float32)],
    )(xp, Wq, Wkbd, bkf, Wvbd, bvf, n1g.reshape(1, F), n1b.reshape(1, F))

    hitp = jnp.full((N, S27), -1, jnp.int32).at[:n].set(hit)
    idxf = jnp.maximum(hitp, 0).reshape(-1)
    kvh = _sc_gatherkv(kvn, idxf)
    vm = jnp.broadcast_to(
        (hitp >= 0).astype(jnp.float32).reshape(N * S27, 1), (N * S27, 8))
    scp = hitp[:, 13]
    qsc = q[jnp.maximum(scp, 0)]
    ic8 = jnp.broadcast_to(
        (scp == jnp.arange(N)).astype(jnp.float32)[:, None], (N, 8))

    y = pl.pallas_call(
        functools.partial(_attn_mlp_body, blk=B),
        grid=(N // B,),
        in_specs=[
            pl.BlockSpec((B, F), lambda i: (i, 0)),
            pl.BlockSpec((B * S27, F), lambda i: (i, 0)),
            pl.BlockSpec((B * S27, F), lambda i: (i, 1)),
            pl.BlockSpec((B * S27, 8), lambda i: (i, 0)),
            pl.BlockSpec((B, 8), lambda i: (i, 0)),
            pl.BlockSpec((B, F), lambda i: (i, 0)),
            _full((1, F)), _full((1, F)),
            _full((NFF, F)), _full((1, NFF)),
            _full((F, NFF)), _full((1, F)),
        ],
        out_specs=pl.BlockSpec((B, F), lambda i: (i, 0)),
        out_shape=jax.ShapeDtypeStruct((N, F), jnp.float32),
    )(qsc, kvh, kvh, vm, ic8, xp,
      n2g.reshape(1, F), n2b.reshape(1, F),
      W1, b1.reshape(1, NFF), W2, b2.reshape(1, F))

    return y[:n]


# speedup vs baseline: 3.6105x; 3.6080x over previous
import functools
import jax
import jax.numpy as jnp
import numpy as np
from jax import lax
from jax.experimental import pallas as pl
from jax.experimental.pallas import tpu as pltpu

GRID = (64, 64, 64)
F = 256
H = 8
D = 32
S27 = 27
NFF = 1024
_EPS = 1e-5
_INVSQRT_D = 1.0 / np.sqrt(D).astype(np.float32)
_HP = jax.lax.Precision.HIGHEST


def _ln(x, g, b):
    mu = jnp.mean(x, axis=-1, keepdims=True)
    xc = x - mu
    var = jnp.mean(xc * xc, axis=-1, keepdims=True)
    return xc * jax.lax.rsqrt(var + _EPS) * g + b


def _qkv_body(x_ref, wq_ref, wk_ref, bk_ref, wv_ref, bv_ref, g_ref, b_ref,
              q_ref, kn_ref, vn_ref):
    x = x_ref[...]
    dn = (((1,), (1,)), ((), ()))
    q_ref[...] = jax.lax.dot_general(x, wq_ref[...], dn,
                                     preferred_element_type=jnp.float32,
                                     precision=_HP)
    k = jax.lax.dot_general(x, wk_ref[...], dn,
                            preferred_element_type=jnp.float32,
                            precision=_HP) + bk_ref[...]
    v = jax.lax.dot_general(x, wv_ref[...], dn,
                            preferred_element_type=jnp.float32,
                            precision=_HP) + bv_ref[...]
    g = g_ref[...]
    b = b_ref[...]
    kn_ref[...] = _ln(k, g, b).astype(jnp.bfloat16)
    vn_ref[...] = _ln(v, g, b).astype(jnp.bfloat16)


def _attn_mlp_body(q_ref, kh_ref, vh_ref, vm_ref, ic_ref, x_ref,
                   g_ref, b_ref, w1_ref, b1_ref, w2_ref, b2_ref, y_ref,
                   *, blk):
    B = blk
    q = q_ref[...]
    kh = kh_ref[...].astype(jnp.float32)
    vh = vh_ref[...].astype(jnp.float32)
    vm = vm_ref[...]

    fi = jax.lax.broadcasted_iota(jnp.int32, (F, H), 0)
    hi = jax.lax.broadcasted_iota(jnp.int32, (F, H), 1)
    seg = (fi // D == hi).astype(jnp.float32)

    prod = jnp.reshape(kh * q[None], (S27 * B, F))
    dots = jax.lax.dot_general(prod, seg, (((1,), (0,)), ((), ())),
                               preferred_element_type=jnp.float32,
                               precision=_HP)
    d3 = jnp.reshape(dots * _INVSQRT_D, (S27, B, H))
    d3 = jnp.where(vm > 0.5, d3, -1e30)
    m = jnp.max(d3, axis=0, keepdims=True)
    e = jnp.exp(d3 - m)
    z = jnp.sum(e, axis=0, keepdims=True)
    a3 = e / z

    a2 = jnp.reshape(a3, (S27 * B, H))
    aw = jax.lax.dot_general(a2, seg, (((1,), (1,)), ((), ())),
                             preferred_element_type=jnp.float32,
                             precision=_HP)
    out = jnp.sum(jnp.reshape(aw, (S27, B, F)) * vh, axis=0)

    ic = ic_ref[...][:, 0:1]
    x = x_ref[...] + out * ic

    h = _ln(x, g_ref[...], b_ref[...])
    h = jax.lax.dot_general(h, w1_ref[...], (((1,), (1,)), ((), ())),
                            preferred_element_type=jnp.float32,
                            precision=_HP) + b1_ref[...]
    h = 0.5 * h * (1.0 + jax.lax.erf(h * np.float32(1.0 / np.sqrt(2.0))))
    h = jax.lax.dot_general(h, w2_ref[...], (((1,), (1,)), ((), ())),
                            preferred_element_type=jnp.float32,
                            precision=_HP) + b2_ref[...]
    y_ref[...] = x + h


def _full(shape):
    return pl.BlockSpec(shape, lambda i: (0,) * len(shape))


def kernel(coords, feats, Wq, Wk, bk, Wv, bv, n1g, n1b, n2g, n2b, W1, b1, W2, b2):
    n = feats.shape[0]
    B = 128
    N = ((n + 511) // 512) * 512

    ids = jnp.arange(1, n + 1, dtype=jnp.int32)
    dense = jnp.zeros(GRID, jnp.int32).at[
        coords[:, 0], coords[:, 1], coords[:, 2]].max(ids)
    padded = jnp.pad(dense, 1)
    offs = jnp.arange(27)
    di, dj, dk = offs // 9, (offs // 3) % 3, offs % 3
    hit = padded[coords[:, 0:1] + di[None],
                 coords[:, 1:2] + dj[None],
                 coords[:, 2:3] + dk[None]] - 1

    hh = jnp.arange(H)
    Mk = jnp.zeros((H, H, D, D), Wk.dtype).at[hh, hh].set(Wk)
    Wkbd = Mk.transpose(0, 2, 1, 3).reshape(F, F)
    Mv = jnp.zeros((H, H, D, D), Wv.dtype).at[hh, hh].set(Wv)
    Wvbd = Mv.transpose(0, 2, 1, 3).reshape(F, F)
    bkf = bk.reshape(1, F)
    bvf = bv.reshape(1, F)

    xp = jnp.zeros((N, F), jnp.float32).at[:n].set(feats)

    Bq = 512
    q, knb, vnb = pl.pallas_call(
        _qkv_body,
        grid=(N // Bq,),
        in_specs=[
            pl.BlockSpec((Bq, F), lambda i: (i, 0)),
            _full((F, F)), _full((F, F)), _full((1, F)),
            _full((F, F)), _full((1, F)),
            _full((1, F)), _full((1, F)),
        ],
        out_specs=[pl.BlockSpec((Bq, F), lambda i: (i, 0))] * 3,
        out_shape=[jax.ShapeDtypeStruct((N, F), jnp.float32),
                   jax.ShapeDtypeStruct((N, F), jnp.bfloat16),
                   jax.ShapeDtypeStruct((N, F), jnp.bfloat16)],
    )(xp, Wq, Wkbd, bkf, Wvbd, bvf, n1g.reshape(1, F), n1b.reshape(1, F))

    hitp = jnp.full((N, S27), -1, jnp.int32).at[:n].set(hit)
    idxt = jnp.maximum(hitp, 0).T.reshape(-1)
    kh2 = knb[idxt].reshape(S27, N, F)
    vh2 = vnb[idxt].reshape(S27, N, F)
    vm3 = jnp.broadcast_to(
        (hitp >= 0).T.astype(jnp.bfloat16)[:, :, None], (S27, N, 8))
    scp = hitp[:, 13]
    qsc = q[jnp.maximum(scp, 0)]
    ic8 = jnp.broadcast_to(
        (scp == jnp.arange(N)).astype(jnp.float32)[:, None], (N, 8))

    y = pl.pallas_call(
        functools.partial(_attn_mlp_body, blk=B),
        grid=(N // B,),
        in_specs=[
            pl.BlockSpec((B, F), lambda i: (i, 0)),
            pl.BlockSpec((S27, B, F), lambda i: (0, i, 0)),
            pl.BlockSpec((S27, B, F), lambda i: (0, i, 0)),
            pl.BlockSpec((S27, B, 8), lambda i: (0, i, 0)),
            pl.BlockSpec((B, 8), lambda i: (i, 0)),
            pl.BlockSpec((B, F), lambda i: (i, 0)),
            _full((1, F)), _full((1, F)),
            _full((NFF, F)), _full((1, NFF)),
            _full((F, NFF)), _full((1, F)),
        ],
        out_specs=pl.BlockSpec((B, F), lambda i: (i, 0)),
        out_shape=jax.ShapeDtypeStruct((N, F), jnp.float32),
    )(qsc, kh2, vh2, vm3, ic8, xp,
      n2g.reshape(1, F), n2b.reshape(1, F),
      W1, b1.reshape(1, NFF), W2, b2.reshape(1, F))

    return y[:n]


# speedup vs baseline: 4.9000x; 1.3572x over previous
import functools
import jax
import jax.numpy as jnp
import numpy as np
from jax import lax
from jax.experimental import pallas as pl
from jax.experimental.pallas import tpu as pltpu

GRID = (64, 64, 64)
F = 256
H = 8
D = 32
S27 = 27
NFF = 1024
_EPS = 1e-5
_INVSQRT_D = 1.0 / np.sqrt(D).astype(np.float32)
_HP = jax.lax.Precision.HIGHEST


def _ln(x, g, b):
    mu = jnp.mean(x, axis=-1, keepdims=True)
    xc = x - mu
    var = jnp.mean(xc * xc, axis=-1, keepdims=True)
    return xc * jax.lax.rsqrt(var + _EPS) * g + b


def _qkv_body(x_ref, wq_ref, wk_ref, bk_ref, wv_ref, bv_ref, g_ref, b_ref,
              q_ref, kv_ref):
    x = x_ref[...]
    dn = (((1,), (1,)), ((), ()))
    q_ref[...] = jax.lax.dot_general(x, wq_ref[...], dn,
                                     preferred_element_type=jnp.float32,
                                     precision=_HP)
    k = jax.lax.dot_general(x, wk_ref[...], dn,
                            preferred_element_type=jnp.float32,
                            precision=_HP) + bk_ref[...]
    v = jax.lax.dot_general(x, wv_ref[...], dn,
                            preferred_element_type=jnp.float32,
                            precision=_HP) + bv_ref[...]
    g = g_ref[...]
    b = b_ref[...]
    kv_ref[:, :F] = _ln(k, g, b).astype(jnp.bfloat16)
    kv_ref[:, F:] = _ln(v, g, b).astype(jnp.bfloat16)


def _attn_mlp_body(q_ref, kv_ref, vm_ref, ic_ref, x_ref,
                   g_ref, b_ref, w1_ref, b1_ref, w2_ref, b2_ref, y_ref,
                   *, blk):
    B = blk
    q = q_ref[...]
    kv = kv_ref[...]
    kh = kv[:, :, :F].astype(jnp.float32)
    vh = kv[:, :, F:].astype(jnp.float32)
    vm = vm_ref[...]

    fi = jax.lax.broadcasted_iota(jnp.int32, (F, H), 0)
    hi = jax.lax.broadcasted_iota(jnp.int32, (F, H), 1)
    seg = (fi // D == hi).astype(jnp.float32)

    prod = jnp.reshape(kh * q[None], (S27 * B, F))
    dots = jax.lax.dot_general(prod, seg, (((1,), (0,)), ((), ())),
                               preferred_element_type=jnp.float32,
                               precision=_HP)
    d3 = jnp.reshape(dots * _INVSQRT_D, (S27, B, H))
    d3 = jnp.where(vm > 0.5, d3, -1e30)
    m = jnp.max(d3, axis=0, keepdims=True)
    e = jnp.exp(d3 - m)
    z = jnp.sum(e, axis=0, keepdims=True)
    a3 = e / z

    a2 = jnp.reshape(a3, (S27 * B, H))
    aw = jax.lax.dot_general(a2, seg, (((1,), (1,)), ((), ())),
                             preferred_element_type=jnp.float32,
                             precision=_HP)
    out = jnp.sum(jnp.reshape(aw, (S27, B, F)) * vh, axis=0)

    ic = ic_ref[...][:, 0:1]
    x = x_ref[...] + out * ic

    h = _ln(x, g_ref[...], b_ref[...])
    h = jax.lax.dot_general(h, w1_ref[...], (((1,), (1,)), ((), ())),
                            preferred_element_type=jnp.float32,
                            precision=_HP) + b1_ref[...]
    h = 0.5 * h * (1.0 + jax.lax.erf(h * np.float32(1.0 / np.sqrt(2.0))))
    h = jax.lax.dot_general(h, w2_ref[...], (((1,), (1,)), ((), ())),
                            preferred_element_type=jnp.float32,
                            precision=_HP) + b2_ref[...]
    y_ref[...] = x + h


def _full(shape):
    return pl.BlockSpec(shape, lambda i: (0,) * len(shape))


def kernel(coords, feats, Wq, Wk, bk, Wv, bv, n1g, n1b, n2g, n2b, W1, b1, W2, b2):
    n = feats.shape[0]
    B = 128
    N = ((n + 511) // 512) * 512

    ids = jnp.arange(1, n + 1, dtype=jnp.int32)
    dense = jnp.zeros(GRID, jnp.int32).at[
        coords[:, 0], coords[:, 1], coords[:, 2]].max(ids)
    padded = jnp.pad(dense, 1)
    offs = jnp.arange(27)
    di, dj, dk = offs // 9, (offs // 3) % 3, offs % 3
    hit = padded[coords[:, 0:1] + di[None],
                 coords[:, 1:2] + dj[None],
                 coords[:, 2:3] + dk[None]] - 1

    hh = jnp.arange(H)
    Mk = jnp.zeros((H, H, D, D), Wk.dtype).at[hh, hh].set(Wk)
    Wkbd = Mk.transpose(0, 2, 1, 3).reshape(F, F)
    Mv = jnp.zeros((H, H, D, D), Wv.dtype).at[hh, hh].set(Wv)
    Wvbd = Mv.transpose(0, 2, 1, 3).reshape(F, F)
    bkf = bk.reshape(1, F)
    bvf = bv.reshape(1, F)

    xp = jnp.zeros((N, F), jnp.float32).at[:n].set(feats)

    Bq = 512
    q, kvb = pl.pallas_call(
        _qkv_body,
        grid=(N // Bq,),
        in_specs=[
            pl.BlockSpec((Bq, F), lambda i: (i, 0)),
            _full((F, F)), _full((F, F)), _full((1, F)),
            _full((F, F)), _full((1, F)),
            _full((1, F)), _full((1, F)),
        ],
        out_specs=[pl.BlockSpec((Bq, F), lambda i: (i, 0)),
                   pl.BlockSpec((Bq, 2 * F), lambda i: (i, 0))],
        out_shape=[jax.ShapeDtypeStruct((N, F), jnp.float32),
                   jax.ShapeDtypeStruct((N, 2 * F), jnp.bfloat16)],
    )(xp, Wq, Wkbd, bkf, Wvbd, bvf, n1g.reshape(1, F), n1b.reshape(1, F))

    hitp = jnp.full((N, S27), -1, jnp.int32).at[:n].set(hit)
    idxt = jnp.maximum(hitp, 0).T.reshape(-1)
    kvh2 = kvb[idxt].reshape(S27, N, 2 * F)
    vm3 = jnp.broadcast_to(
        (hitp >= 0).T.astype(jnp.bfloat16)[:, :, None], (S27, N, 8))
    scp = hitp[:, 13]
    qsc = q[jnp.maximum(scp, 0)]
    ic8 = jnp.broadcast_to(
        (scp == jnp.arange(N)).astype(jnp.float32)[:, None], (N, 8))

    y = pl.pallas_call(
        functools.partial(_attn_mlp_body, blk=B),
        grid=(N // B,),
        in_specs=[
            pl.BlockSpec((B, F), lambda i: (i, 0)),
            pl.BlockSpec((S27, B, 2 * F), lambda i: (0, i, 0)),
            pl.BlockSpec((S27, B, 8), lambda i: (0, i, 0)),
            pl.BlockSpec((B, 8), lambda i: (i, 0)),
            pl.BlockSpec((B, F), lambda i: (i, 0)),
            _full((1, F)), _full((1, F)),
            _full((NFF, F)), _full((1, NFF)),
            _full((F, NFF)), _full((1, F)),
        ],
        out_specs=pl.BlockSpec((B, F), lambda i: (i, 0)),
        out_shape=jax.ShapeDtypeStruct((N, F), jnp.float32),
    )(qsc, kvh2, vm3, ic8, xp,
      n2g.reshape(1, F), n2b.reshape(1, F),
      W1, b1.reshape(1, NFF), W2, b2.reshape(1, F))

    return y[:n]


# speedup vs baseline: 5.1182x; 1.0445x over previous
import functools
import jax
import jax.numpy as jnp
import numpy as np
from jax import lax
from jax.experimental import pallas as pl
from jax.experimental.pallas import tpu as pltpu

GRID = (64, 64, 64)
F = 256
H = 8
D = 32
S27 = 27
NFF = 1024
_EPS = 1e-5
_INVSQRT_D = 1.0 / np.sqrt(D).astype(np.float32)
_HP = jax.lax.Precision.HIGHEST


def _ln(x, g, b):
    mu = jnp.mean(x, axis=-1, keepdims=True)
    xc = x - mu
    var = jnp.mean(xc * xc, axis=-1, keepdims=True)
    return xc * jax.lax.rsqrt(var + _EPS) * g + b


def _qkv_body(x_ref, wq_ref, wk_ref, bk_ref, wv_ref, bv_ref, g_ref, b_ref,
              q_ref, kv_ref):
    x = x_ref[...]
    dn = (((1,), (1,)), ((), ()))
    q_ref[...] = jax.lax.dot_general(x, wq_ref[...], dn,
                                     preferred_element_type=jnp.float32,
                                     precision=_HP)
    k = jax.lax.dot_general(x, wk_ref[...], dn,
                            preferred_element_type=jnp.float32,
                            precision=_HP) + bk_ref[...]
    v = jax.lax.dot_general(x, wv_ref[...], dn,
                            preferred_element_type=jnp.float32,
                            precision=_HP) + bv_ref[...]
    g = g_ref[...]
    b = b_ref[...]
    kv_ref[:, :F] = _ln(k, g, b).astype(jnp.bfloat16)
    kv_ref[:, F:] = _ln(v, g, b).astype(jnp.bfloat16)


def _attn_mlp_body(q_ref, kv_ref, vm_ref, ic_ref, x_ref,
                   g_ref, b_ref, w1_ref, b1_ref, w2_ref, b2_ref, y_ref,
                   *, blk):
    B = blk
    q = q_ref[...]
    kv = kv_ref[...]
    kh = kv[:, :, :F].astype(jnp.float32)
    vh = kv[:, :, F:].astype(jnp.float32)
    vm = vm_ref[...]

    fi = jax.lax.broadcasted_iota(jnp.int32, (F, H), 0)
    hi = jax.lax.broadcasted_iota(jnp.int32, (F, H), 1)
    seg = (fi // D == hi).astype(jnp.float32)

    prod = jnp.reshape(kh * q[None], (S27 * B, F))
    dots = jax.lax.dot_general(prod, seg, (((1,), (0,)), ((), ())),
                               preferred_element_type=jnp.float32,
                               precision=_HP)
    d3 = jnp.reshape(dots * _INVSQRT_D, (S27, B, H))
    d3 = jnp.where(vm > 0.5, d3, -1e30)
    m = jnp.max(d3, axis=0, keepdims=True)
    e = jnp.exp(d3 - m)
    z = jnp.sum(e, axis=0, keepdims=True)
    a3 = e / z

    a2 = jnp.reshape(a3, (S27 * B, H))
    aw = jax.lax.dot_general(a2, seg, (((1,), (1,)), ((), ())),
                             preferred_element_type=jnp.float32,
                             precision=_HP)
    out = jnp.sum(jnp.reshape(aw, (S27, B, F)) * vh, axis=0)

    ic = ic_ref[...][:, 0:1]
    x = x_ref[...] + out * ic

    h = _ln(x, g_ref[...], b_ref[...])
    h = jax.lax.dot_general(h, w1_ref[...], (((1,), (1,)), ((), ())),
                            preferred_element_type=jnp.float32,
                            precision=_HP) + b1_ref[...]
    h = 0.5 * h * (1.0 + jax.lax.erf(h * np.float32(1.0 / np.sqrt(2.0))))
    h = jax.lax.dot_general(h, w2_ref[...], (((1,), (1,)), ((), ())),
                            preferred_element_type=jnp.float32,
                            precision=_HP) + b2_ref[...]
    y_ref[...] = x + h


def _full(shape):
    return pl.BlockSpec(shape, lambda i: (0,) * len(shape))


def kernel(coords, feats, Wq, Wk, bk, Wv, bv, n1g, n1b, n2g, n2b, W1, b1, W2, b2):
    n = feats.shape[0]
    B = 256
    N = ((n + 511) // 512) * 512

    ids = jnp.arange(1, n + 1, dtype=jnp.int32)
    dense = jnp.zeros(GRID, jnp.int32).at[
        coords[:, 0], coords[:, 1], coords[:, 2]].max(ids)
    padded = jnp.pad(dense, 1)
    offs = jnp.arange(27)
    di, dj, dk = offs // 9, (offs // 3) % 3, offs % 3
    hit = padded[coords[:, 0:1] + di[None],
                 coords[:, 1:2] + dj[None],
                 coords[:, 2:3] + dk[None]] - 1

    hh = jnp.arange(H)
    Mk = jnp.zeros((H, H, D, D), Wk.dtype).at[hh, hh].set(Wk)
    Wkbd = Mk.transpose(0, 2, 1, 3).reshape(F, F)
    Mv = jnp.zeros((H, H, D, D), Wv.dtype).at[hh, hh].set(Wv)
    Wvbd = Mv.transpose(0, 2, 1, 3).reshape(F, F)
    bkf = bk.reshape(1, F)
    bvf = bv.reshape(1, F)

    xp = jnp.zeros((N, F), jnp.float32).at[:n].set(feats)

    Bq = 512
    q, kvb = pl.pallas_call(
        _qkv_body,
        grid=(N // Bq,),
        in_specs=[
            pl.BlockSpec((Bq, F), lambda i: (i, 0)),
            _full((F, F)), _full((F, F)), _full((1, F)),
            _full((F, F)), _full((1, F)),
            _full((1, F)), _full((1, F)),
        ],
        out_specs=[pl.BlockSpec((Bq, F), lambda i: (i, 0)),
                   pl.BlockSpec((Bq, 2 * F), lambda i: (i, 0))],
        out_shape=[jax.ShapeDtypeStruct((N, F), jnp.float32),
                   jax.ShapeDtypeStruct((N, 2 * F), jnp.bfloat16)],
    )(xp, Wq, Wkbd, bkf, Wvbd, bvf, n1g.reshape(1, F), n1b.reshape(1, F))

    hitp = jnp.full((N, S27), -1, jnp.int32).at[:n].set(hit)
    idxt = jnp.maximum(hitp, 0).T.reshape(-1)
    kvh2 = kvb[idxt].reshape(S27, N, 2 * F)
    vm3 = jnp.broadcast_to(
        (hitp >= 0).T.astype(jnp.bfloat16)[:, :, None], (S27, N, 8))
    scp = hitp[:, 13]
    qsc = q[jnp.maximum(scp, 0)]
    ic8 = jnp.broadcast_to(
        (scp == jnp.arange(N)).astype(jnp.float32)[:, None], (N, 8))

    y = pl.pallas_call(
        functools.partial(_attn_mlp_body, blk=B),
        grid=(N // B,),
        in_specs=[
            pl.BlockSpec((B, F), lambda i: (i, 0)),
            pl.BlockSpec((S27, B, 2 * F), lambda i: (0, i, 0)),
            pl.BlockSpec((S27, B, 8), lambda i: (0, i, 0)),
            pl.BlockSpec((B, 8), lambda i: (i, 0)),
            pl.BlockSpec((B, F), lambda i: (i, 0)),
            _full((1, F)), _full((1, F)),
            _full((NFF, F)), _full((1, NFF)),
            _full((F, NFF)), _full((1, F)),
        ],
        out_specs=pl.BlockSpec((B, F), lambda i: (i, 0)),
        out_shape=jax.ShapeDtypeStruct((N, F), jnp.float32),
    )(qsc, kvh2, vm3, ic8, xp,
      n2g.reshape(1, F), n2b.reshape(1, F),
      W1, b1.reshape(1, NFF), W2, b2.reshape(1, F))

    return y[:n]
